# Initial kernel scaffold; baseline (speedup 1.0000x reference)
import jax
import jax.numpy as jnp
from jax.experimental import pallas as pl


def kernel(sequence, segment_label, token_table, segment_table):
    raise NotImplementedError("write your pallas kernel here")


# speedup vs baseline: 1.2577x; 1.2577x over previous
import functools

import numpy as np
import jax
import jax.numpy as jnp
from jax import lax
from jax.experimental import pallas as pl
from jax.experimental.pallas import tpu as pltpu
from jax.experimental.pallas import tpu_sc as plsc

EMBED = 64
SEQ = 200
BATCH = 1024
MAX_LEN = 512

NC, NS = 2, 16
NW = NC * NS
N = BATCH * SEQ
NPW = N // NW
SUB = 128
CH = 640
NCHUNK = NPW // CH
KSUB = CH // SUB
ROWS_J = NPW // SUB


def _make_pe_np(max_len, d_model):
    position = np.arange(max_len, dtype=np.float32)[:, None]
    div_term = np.exp(
        np.arange(0, d_model, 2, dtype=np.float32) * -(np.log(10000.0) / d_model)
    )
    pe = np.zeros((max_len, d_model), dtype=np.float32)
    pe[:, 0::2] = np.sin(position * div_term)
    pe[:, 1::2] = np.cos(position * div_term)
    return pe


_PE = _make_pe_np(MAX_LEN, EMBED)[:SEQ]


def _sc_body(seq1d, seg1d, tok, comb, out,
             idx_v, seg_v, cidx_v, rows_v, crows_v, sem_t, sem_c):
    c = lax.axis_index("c")
    s = lax.axis_index("s")
    wid = s * NC + c
    base = wid * NPW

    pltpu.sync_copy(seq1d.at[pl.ds(base, NPW)], idx_v)
    pltpu.sync_copy(seg1d.at[pl.ds(base, NPW)], seg_v)

    lane = lax.iota(jnp.int32, 16)

    @pl.loop(0, NPW // 16)
    def _cidx(g):
        off = g * 16
        pos = lax.rem(off + lane, SEQ)
        cidx_v[pl.ds(off, 16)] = seg_v[pl.ds(off, 16)] * SEQ + pos

    @pl.loop(0, NCHUNK)
    def _chunk(ci):
        cb = ci * CH
        descs = []
        for k in range(KSUB):
            descs.append(pltpu.async_copy(
                tok.at[idx_v.at[pl.ds(cb + k * SUB, SUB)]],
                rows_v.at[pl.ds(k * SUB, SUB)], sem_t))
            descs.append(pltpu.async_copy(
                comb.at[cidx_v.at[pl.ds(cb + k * SUB, SUB)]],
                crows_v.at[pl.ds(k * SUB, SUB)], sem_c))
        for d in descs:
            d.wait()

        @pl.loop(0, CH)
        def _add(r):
            for u in range(EMBED // 16):
                sl = pl.ds(u * 16, 16)
                rows_v[r, sl] = rows_v[r, sl] + crows_v[r, sl]

        pltpu.sync_copy(rows_v, out.at[pl.ds(base + ci * CH, CH)])


@functools.partial(jax.jit, static_argnames=("interpret",))
def _sc_call(seq1d, seg1d, tok, comb, interpret=False):
    mesh = plsc.VectorSubcoreMesh(
        core_axis_name="c", subcore_axis_name="s",
        num_cores=NC, num_subcores=NS)
    return pl.kernel(
        _sc_body,
        out_type=jax.ShapeDtypeStruct((N, EMBED), jnp.float32),
        mesh=mesh,
        scratch_types=[
            pltpu.VMEM((NPW,), jnp.int32),
            pltpu.VMEM((NPW,), jnp.int32),
            pltpu.VMEM((NPW,), jnp.int32),
            pltpu.VMEM((CH, EMBED), jnp.float32),
            pltpu.VMEM((CH, EMBED), jnp.float32),
            pltpu.SemaphoreType.DMA,
            pltpu.SemaphoreType.DMA,
        ],
        compiler_params=pltpu.CompilerParams(use_tc_tiling_on_sc=False),
        interpret=interpret,
    )(seq1d, seg1d, tok, comb)


def kernel(sequence, segment_label, token_table, segment_table):
    b, s = sequence.shape
    seq1d = sequence.reshape(N).astype(jnp.int32)
    seg1d = segment_label.reshape(N).astype(jnp.int32)
    pe = jnp.asarray(_PE)
    comb = (segment_table[:, None, :] + pe[None, :, :]).reshape(3 * SEQ, EMBED)
    out = _sc_call(seq1d, seg1d, token_table, comb)
    return out.reshape(b, s, EMBED)
